# baseline (speedup 1.0000x reference)
import functools

import jax
import jax.numpy as jnp
from jax import lax
from jax.experimental import pallas as pl
from jax.experimental.pallas import tpu as pltpu
from jax.experimental.pallas import tpu_sc as plsc

B = 1
S = 2048
HIDDEN = 2048
FFN = 512
N_ROUTED = 64
N_EXP = 80
EPAD = 128
TOPK = 2
SCALE = 1.0
EPS = 1e-20

TM = 64
M_PAD = 8192
N_TILES = M_PAD // TM
S_TILE = 256


def _router_body(x_ref, rwt_ref, bias_ref, out_ref, xb_ref):
    xb_ref[...] = x_ref[...].astype(jnp.bfloat16)
    logits = jax.lax.dot_general(
        x_ref[...], rwt_ref[...], (((1,), (0,)), ((), ())),
        preferred_element_type=jnp.float32)
    scores = jax.nn.sigmoid(logits)
    c = scores + bias_ref[0:1, :]
    iota = jax.lax.broadcasted_iota(jnp.int32, (S_TILE, EPAD), 1)
    m0 = jnp.max(c, axis=1, keepdims=True)
    a0 = jnp.min(jnp.where(c == m0, iota, EPAD), axis=1, keepdims=True)
    c1 = jnp.where(iota == a0, -1e30, c)
    m1 = jnp.max(c1, axis=1, keepdims=True)
    a1 = jnp.min(jnp.where(c1 == m1, iota, EPAD), axis=1, keepdims=True)
    s0 = jnp.sum(jnp.where(iota == a0, scores, 0.0), axis=1, keepdims=True)
    s1 = jnp.sum(jnp.where(iota == a1, scores, 0.0), axis=1, keepdims=True)
    den = s0 + s1 + EPS
    w0 = s0 / den * SCALE
    w1 = s1 / den * SCALE
    z0 = a0 >= N_ROUTED
    z1 = a1 >= N_ROUTED
    rw0 = jnp.where(z0, 0.0, w0)
    rw1 = jnp.where(z1, 0.0, w1)
    zw = jnp.where(z0, w0, 0.0) + jnp.where(z1, w1, 0.0)
    e0 = jnp.where(z0, N_ROUTED, a0).astype(jnp.float32)
    e1 = jnp.where(z1, N_ROUTED, a1).astype(jnp.float32)
    li = jax.lax.broadcasted_iota(jnp.int32, (S_TILE, EPAD), 1)
    out = jnp.where(li == 0, e0, 0.0)
    out = jnp.where(li == 1, e1, out)
    out = jnp.where(li == 2, rw0, out)
    out = jnp.where(li == 3, rw1, out)
    out = jnp.where(li == 4, zw, out)
    out_ref[...] = out


def _router(x, rwt_pad, bias_pad):
    return pl.pallas_call(
        _router_body,
        grid=(S // S_TILE,),
        in_specs=[
            pl.BlockSpec((S_TILE, HIDDEN), lambda i: (i, 0)),
            pl.BlockSpec((HIDDEN, EPAD), lambda i: (0, 0)),
            pl.BlockSpec((8, EPAD), lambda i: (0, 0)),
        ],
        out_specs=[pl.BlockSpec((S_TILE, EPAD), lambda i: (i, 0)),
                   pl.BlockSpec((S_TILE, HIDDEN), lambda i: (i, 0))],
        out_shape=[jax.ShapeDtypeStruct((S, EPAD), jnp.float32),
                   jax.ShapeDtypeStruct((S, HIDDEN), jnp.bfloat16)],
    )(x, rwt_pad, bias_pad)


def _gmm_body(te_ref, xs_ref, wg_ref, wu_ref, wd_ref, ys_ref):
    t = pl.program_id(0)
    e = te_ref[t]

    @pl.when(e >= 0)
    def _():
        xt = xs_ref[...]
        wg = wg_ref[0].astype(jnp.bfloat16)
        wu = wu_ref[0].astype(jnp.bfloat16)
        wd = wd_ref[0].astype(jnp.bfloat16)
        g = jax.lax.dot_general(xt, wg, (((1,), (1,)), ((), ())),
                                preferred_element_type=jnp.float32)
        u = jax.lax.dot_general(xt, wu, (((1,), (1,)), ((), ())),
                                preferred_element_type=jnp.float32)
        h = (g * jax.nn.sigmoid(g) * u).astype(jnp.bfloat16)
        y = jax.lax.dot_general(h, wd, (((1,), (1,)), ((), ())),
                                preferred_element_type=jnp.float32)
        ys_ref[...] = y

    @pl.when(jnp.logical_and(e < 0, t == 0))
    def _():
        ys_ref[...] = jnp.zeros_like(ys_ref)


def _gmm(te, xs, w_gate, w_up, w_down):
    def clamp(e):
        return jnp.clip(e, 0, N_ROUTED - 1)

    grid_spec = pltpu.PrefetchScalarGridSpec(
        num_scalar_prefetch=1,
        grid=(N_TILES,),
        in_specs=[
            pl.BlockSpec((TM, HIDDEN), lambda t, te: (t, 0)),
            pl.BlockSpec((1, FFN, HIDDEN), lambda t, te: (clamp(te[t]), 0, 0)),
            pl.BlockSpec((1, FFN, HIDDEN), lambda t, te: (clamp(te[t]), 0, 0)),
            pl.BlockSpec((1, HIDDEN, FFN), lambda t, te: (clamp(te[t]), 0, 0)),
        ],
        out_specs=pl.BlockSpec((TM, HIDDEN), lambda t, te: (t, 0)),
    )
    return pl.pallas_call(
        _gmm_body,
        grid_spec=grid_spec,
        out_shape=jax.ShapeDtypeStruct((M_PAD, HIDDEN), jnp.float32),
    )(te, xs, w_gate, w_up, w_down)


NW = 32
G_ROWS = M_PAD // NW
GCH = 32
_SC_MESH = dict(core_axis_name="c", subcore_axis_name="s")


def _sc_wid():
    return lax.axis_index("s") * 2 + lax.axis_index("c")


HID2 = HIDDEN // 2


def _gather_rows(xb32, tor2, rmeta):
    NCH = G_ROWS // GCH

    @functools.partial(
        pl.kernel,
        out_type=jax.ShapeDtypeStruct((M_PAD, HID2), jnp.int32),
        mesh=plsc.VectorSubcoreMesh(**_SC_MESH),
        scratch_types=[
            pltpu.VMEM((NCH, GCH), jnp.int32),
            pltpu.VMEM((16,), jnp.int32),
            pltpu.VMEM((2, GCH, HID2), jnp.int32),
            pltpu.SemaphoreType.DMA,
            pltpu.SemaphoreType.DMA,
            pltpu.SemaphoreType.DMA,
            pltpu.SemaphoreType.DMA,
        ],
    )
    def k(x_hbm, tor_hbm, rm_hbm, xs_hbm, idx_v, rm_v, rows_v, g0, g1, w0, w1):
        wid = _sc_wid()
        gsem = (g0, g1)
        wsem = (w0, w1)
        pltpu.sync_copy(tor_hbm.at[pl.ds(wid * NCH, NCH)], idx_v)
        pltpu.sync_copy(rm_hbm, rm_v)
        rtot = rm_v[pl.ds(0, 16)][0]

        def pred(c):
            return wid * G_ROWS + c * GCH < rtot

        @pl.when(pred(0))
        def _():
            pltpu.async_copy(x_hbm.at[idx_v.at[0]], rows_v.at[0], gsem[0])

        for c in range(NCH):
            b = c % 2
            nb = (c + 1) % 2
            if c + 1 < NCH:
                @pl.when(pred(c + 1))
                def _(c=c, b=b, nb=nb):
                    if c >= 1:
                        pltpu.make_async_copy(rows_v.at[nb],
                                              xs_hbm.at[pl.ds(0, GCH)],
                                              wsem[nb]).wait()
                    pltpu.async_copy(x_hbm.at[idx_v.at[c + 1]],
                                     rows_v.at[nb], gsem[nb])

            @pl.when(pred(c))
            def _(c=c, b=b):
                pltpu.make_async_copy(x_hbm.at[idx_v.at[c]], rows_v.at[b],
                                      gsem[b]).wait()
                pltpu.async_copy(
                    rows_v.at[b],
                    xs_hbm.at[pl.ds(wid * G_ROWS + c * GCH, GCH)],
                    wsem[b])

        for b in range(2):
            last = NCH - 2 + b
            @pl.when(pred(last))
            def _(b=b):
                pltpu.make_async_copy(rows_v.at[b],
                                      xs_hbm.at[pl.ds(0, GCH)],
                                      wsem[b]).wait()

    return k(xb32, tor2, rmeta)


C_TOK = 4
C_PER_W = S // NW
N_CH = C_PER_W // C_TOK
_UNROLL = 4


def _combine(x, ys, p02, p12, rw0, rw1, zw):
    @functools.partial(
        pl.kernel,
        out_type=jax.ShapeDtypeStruct((S, HIDDEN), jnp.float32),
        mesh=plsc.VectorSubcoreMesh(**_SC_MESH),
        scratch_types=[
            pltpu.VMEM((N_CH, C_TOK), jnp.int32),
            pltpu.VMEM((N_CH, C_TOK), jnp.int32),
            pltpu.VMEM((C_PER_W,), jnp.float32),
            pltpu.VMEM((C_PER_W,), jnp.float32),
            pltpu.VMEM((C_PER_W,), jnp.float32),
            pltpu.VMEM((2, C_TOK, HIDDEN), jnp.float32),
            pltpu.VMEM((2, C_TOK, HIDDEN), jnp.float32),
            pltpu.VMEM((2, C_TOK, HIDDEN), jnp.float32),
            pltpu.VMEM((2, C_TOK, HIDDEN), jnp.float32),
            pltpu.SemaphoreType.DMA,
            pltpu.SemaphoreType.DMA,
            pltpu.SemaphoreType.DMA,
            pltpu.SemaphoreType.DMA,
        ],
    )
    def k(x_hbm, ys_hbm, p0_hbm, p1_hbm, rw0_hbm, rw1_hbm, zw_hbm, out_hbm,
          p0_v, p1_v, rw0_v, rw1_v, zw_v, x_b, y0_b, y1_b, o_b,
          gs0, gs1, ws0, ws1):
        gsem = (gs0, gs1)
        wsem = (ws0, ws1)
        wid = _sc_wid()
        base = wid * C_PER_W
        pltpu.sync_copy(p0_hbm.at[pl.ds(wid * N_CH, N_CH)], p0_v)
        pltpu.sync_copy(p1_hbm.at[pl.ds(wid * N_CH, N_CH)], p1_v)
        pltpu.sync_copy(rw0_hbm.at[pl.ds(base, C_PER_W)], rw0_v)
        pltpu.sync_copy(rw1_hbm.at[pl.ds(base, C_PER_W)], rw1_v)
        pltpu.sync_copy(zw_hbm.at[pl.ds(base, C_PER_W)], zw_v)

        def fire(c):
            b = c % 2
            pltpu.async_copy(ys_hbm.at[p0_v.at[c]], y0_b.at[b], gsem[b])
            pltpu.async_copy(ys_hbm.at[p1_v.at[c]], y1_b.at[b], gsem[b])
            pltpu.async_copy(x_hbm.at[pl.ds(base + c * C_TOK, C_TOK)],
                             x_b.at[b], gsem[b])

        def drain_in(c):
            b = c % 2
            pltpu.make_async_copy(ys_hbm.at[p0_v.at[c]], y0_b.at[b],
                                  gsem[b]).wait()
            pltpu.make_async_copy(ys_hbm.at[p1_v.at[c]], y1_b.at[b],
                                  gsem[b]).wait()
            pltpu.make_async_copy(x_hbm.at[pl.ds(base, C_TOK)], x_b.at[b],
                                  gsem[b]).wait()

        fire(0)
        for c in range(N_CH):
            b = c % 2
            if c >= 2:
                pltpu.make_async_copy(o_b.at[b], out_hbm.at[pl.ds(0, C_TOK)],
                                      wsem[b]).wait()
            if c + 1 < N_CH:
                fire(c + 1)
            drain_in(c)
            for j in range(C_TOK):
                q = c * C_TOK + j
                qs, ql = (q // 16) * 16, q % 16
                w0 = rw0_v[pl.ds(qs, 16)][ql]
                w1 = rw1_v[pl.ds(qs, 16)][ql]
                wz = zw_v[pl.ds(qs, 16)][ql]

                def lane(kk, carry, b=b, j=j, w0=w0, w1=w1, wz=wz):
                    for u in range(_UNROLL):
                        sl = pl.ds(kk * (16 * _UNROLL) + u * 16, 16)
                        o_b[b, j, sl] = (wz * x_b[b, j, sl]
                                         + w0 * y0_b[b, j, sl]
                                         + w1 * y1_b[b, j, sl])
                    return carry

                lax.fori_loop(0, HIDDEN // (16 * _UNROLL), lane, 0)
            pltpu.async_copy(o_b.at[b],
                             out_hbm.at[pl.ds(base + c * C_TOK, C_TOK)],
                             wsem[b])
        pltpu.make_async_copy(o_b.at[0], out_hbm.at[pl.ds(0, C_TOK)],
                              wsem[0]).wait()
        pltpu.make_async_copy(o_b.at[1], out_hbm.at[pl.ds(0, C_TOK)],
                              wsem[1]).wait()

    return k(x, ys, p02, p12, rw0, rw1, zw)


def _dispatch(e0, e1):
    keys = jnp.concatenate([e0, e1])
    order = jnp.argsort(keys)
    ks = keys[order]
    first = jnp.searchsorted(ks, ks, side="left")
    rank_sorted = jnp.arange(2 * S, dtype=jnp.int32) - first.astype(jnp.int32)
    rank = jnp.zeros(2 * S, jnp.int32).at[order].set(rank_sorted)
    counts = jnp.bincount(keys, length=N_ROUTED + 1).astype(jnp.int32)
    rc = ((counts[:N_ROUTED] + TM - 1) // TM) * TM
    offsets = jnp.concatenate([jnp.zeros(1, jnp.int32), jnp.cumsum(rc)])
    valid = keys < N_ROUTED
    pos = jnp.where(valid, offsets[jnp.minimum(keys, N_ROUTED - 1)] + rank, 0)
    token_of_row = jnp.zeros(M_PAD, jnp.int32).at[
        jnp.where(valid, pos, M_PAD - 1)].set(
        jnp.where(valid, jnp.arange(2 * S, dtype=jnp.int32) % S, 0))
    tile_start = offsets // TM
    t_iota = jnp.arange(N_TILES)
    te = jnp.sum(tile_start[None, 1:N_ROUTED] <= t_iota[:, None], axis=1)
    te = jnp.where(t_iota < tile_start[N_ROUTED], te, -1).astype(jnp.int32)
    rtot = jnp.broadcast_to(offsets[N_ROUTED], (16,)).astype(jnp.int32)
    return pos, token_of_row, te, rtot


def kernel(hidden_states, router_weight, e_score_correction_bias, w_gate, w_up, w_down):
    x = hidden_states.reshape(-1, HIDDEN).astype(jnp.float32)
    rwt_pad = jnp.zeros((HIDDEN, EPAD), jnp.float32).at[:, :N_EXP].set(router_weight.T)
    bias_pad = jnp.full((8, EPAD), -1e30, jnp.float32).at[:, :N_EXP].set(
        e_score_correction_bias[None, :])
    r, xb = _router(x, rwt_pad, bias_pad)
    e0 = r[:, 0].astype(jnp.int32)
    e1 = r[:, 1].astype(jnp.int32)
    rw0 = r[:, 2]
    rw1 = r[:, 3]
    zw = r[:, 4]

    pos, token_of_row, te, rtot = _dispatch(e0, e1)
    xb32 = jax.lax.bitcast_convert_type(xb.reshape(S, HID2, 2), jnp.int32)
    xs32 = _gather_rows(xb32, token_of_row.reshape(M_PAD // GCH, GCH), rtot)
    xs = jax.lax.bitcast_convert_type(
        xs32, jnp.bfloat16).reshape(M_PAD, HIDDEN)
    ys = _gmm(te, xs, w_gate, w_up, w_down)
    p02 = pos[:S].reshape(S // C_TOK, C_TOK)
    p12 = pos[S:].reshape(S // C_TOK, C_TOK)
    out = _combine(x, ys, p02, p12, rw0, rw1, zw)
    return out.reshape(B, S, HIDDEN)


# speedup vs baseline: 1.4025x; 1.4025x over previous
import functools

import jax
import jax.numpy as jnp
from jax import lax
from jax.experimental import pallas as pl
from jax.experimental.pallas import tpu as pltpu
from jax.experimental.pallas import tpu_sc as plsc

B = 1
S = 2048
HIDDEN = 2048
FFN = 512
N_ROUTED = 64
N_EXP = 80
EPAD = 128
TOPK = 2
SCALE = 1.0
EPS = 1e-20

HID2 = HIDDEN // 2
TM = 64
M_PAD = 8192
N_TILES = M_PAD // TM
S_TILE = 256


def _pack2(x):
    c = x.shape[1] // 2
    lo = jax.lax.bitcast_convert_type(x[:, :c].astype(jnp.bfloat16), jnp.uint16)
    hi = jax.lax.bitcast_convert_type(x[:, c:].astype(jnp.bfloat16), jnp.uint16)
    w = lo.astype(jnp.uint32) | (hi.astype(jnp.uint32) << 16)
    return jax.lax.bitcast_convert_type(w, jnp.int32)


def _unpack2(w):
    u = jax.lax.bitcast_convert_type(w, jnp.uint32)
    lo = jax.lax.bitcast_convert_type((u & 0xFFFF).astype(jnp.uint16),
                                      jnp.bfloat16)
    hi = jax.lax.bitcast_convert_type((u >> 16).astype(jnp.uint16),
                                      jnp.bfloat16)
    return jnp.concatenate([lo, hi], axis=1)


def _router_body(x_ref, rwt_ref, bias_ref, out_ref, xb_ref):
    xb_ref[...] = _pack2(x_ref[...])
    logits = jax.lax.dot_general(
        x_ref[...], rwt_ref[...], (((1,), (0,)), ((), ())),
        preferred_element_type=jnp.float32)
    scores = jax.nn.sigmoid(logits)
    c = scores + bias_ref[0:1, :]
    iota = jax.lax.broadcasted_iota(jnp.int32, (S_TILE, EPAD), 1)
    m0 = jnp.max(c, axis=1, keepdims=True)
    a0 = jnp.min(jnp.where(c == m0, iota, EPAD), axis=1, keepdims=True)
    c1 = jnp.where(iota == a0, -1e30, c)
    m1 = jnp.max(c1, axis=1, keepdims=True)
    a1 = jnp.min(jnp.where(c1 == m1, iota, EPAD), axis=1, keepdims=True)
    s0 = jnp.sum(jnp.where(iota == a0, scores, 0.0), axis=1, keepdims=True)
    s1 = jnp.sum(jnp.where(iota == a1, scores, 0.0), axis=1, keepdims=True)
    den = s0 + s1 + EPS
    w0 = s0 / den * SCALE
    w1 = s1 / den * SCALE
    z0 = a0 >= N_ROUTED
    z1 = a1 >= N_ROUTED
    rw0 = jnp.where(z0, 0.0, w0)
    rw1 = jnp.where(z1, 0.0, w1)
    zw = jnp.where(z0, w0, 0.0) + jnp.where(z1, w1, 0.0)
    e0 = jnp.where(z0, N_ROUTED, a0).astype(jnp.float32)
    e1 = jnp.where(z1, N_ROUTED, a1).astype(jnp.float32)
    li = jax.lax.broadcasted_iota(jnp.int32, (S_TILE, EPAD), 1)
    out = jnp.where(li == 0, e0, 0.0)
    out = jnp.where(li == 1, e1, out)
    out = jnp.where(li == 2, rw0, out)
    out = jnp.where(li == 3, rw1, out)
    out = jnp.where(li == 4, zw, out)
    out_ref[...] = out


def _router(x, rwt_pad, bias_pad):
    return pl.pallas_call(
        _router_body,
        grid=(S // S_TILE,),
        in_specs=[
            pl.BlockSpec((S_TILE, HIDDEN), lambda i: (i, 0)),
            pl.BlockSpec((HIDDEN, EPAD), lambda i: (0, 0)),
            pl.BlockSpec((8, EPAD), lambda i: (0, 0)),
        ],
        out_specs=[pl.BlockSpec((S_TILE, EPAD), lambda i: (i, 0)),
                   pl.BlockSpec((S_TILE, HID2), lambda i: (i, 0))],
        out_shape=[jax.ShapeDtypeStruct((S, EPAD), jnp.float32),
                   jax.ShapeDtypeStruct((S, HID2), jnp.int32)],
    )(x, rwt_pad, bias_pad)


def _gmm_body(te_ref, xs_ref, wg_ref, wu_ref, wd_ref, ys_ref):
    t = pl.program_id(0)
    e = te_ref[t]

    @pl.when(e >= 0)
    def _():
        xt = _unpack2(xs_ref[...])
        wg = wg_ref[0].astype(jnp.bfloat16)
        wu = wu_ref[0].astype(jnp.bfloat16)
        wd = wd_ref[0].astype(jnp.bfloat16)
        g = jax.lax.dot_general(xt, wg, (((1,), (1,)), ((), ())),
                                preferred_element_type=jnp.float32)
        u = jax.lax.dot_general(xt, wu, (((1,), (1,)), ((), ())),
                                preferred_element_type=jnp.float32)
        h = (g * jax.nn.sigmoid(g) * u).astype(jnp.bfloat16)
        y = jax.lax.dot_general(h, wd, (((1,), (1,)), ((), ())),
                                preferred_element_type=jnp.float32)
        ys_ref[...] = y

    @pl.when(jnp.logical_and(e < 0, t == 0))
    def _():
        ys_ref[...] = jnp.zeros_like(ys_ref)


def _gmm(te, xs, w_gate, w_up, w_down):
    def clamp(e):
        return jnp.clip(e, 0, N_ROUTED - 1)

    grid_spec = pltpu.PrefetchScalarGridSpec(
        num_scalar_prefetch=1,
        grid=(N_TILES,),
        in_specs=[
            pl.BlockSpec((TM, HID2), lambda t, te: (t, 0)),
            pl.BlockSpec((1, FFN, HIDDEN), lambda t, te: (clamp(te[t]), 0, 0)),
            pl.BlockSpec((1, FFN, HIDDEN), lambda t, te: (clamp(te[t]), 0, 0)),
            pl.BlockSpec((1, HIDDEN, FFN), lambda t, te: (clamp(te[t]), 0, 0)),
        ],
        out_specs=pl.BlockSpec((TM, HIDDEN), lambda t, te: (t, 0)),
    )
    return pl.pallas_call(
        _gmm_body,
        grid_spec=grid_spec,
        out_shape=jax.ShapeDtypeStruct((M_PAD, HIDDEN), jnp.float32),
    )(te, xs, w_gate, w_up, w_down)


NW = 32
G_ROWS = M_PAD // NW
GCH = 32
_SC_MESH = dict(core_axis_name="c", subcore_axis_name="s")


def _sc_wid():
    return lax.axis_index("s") * 2 + lax.axis_index("c")


def _gather_rows(xb32, tor2, rmeta):
    NCH = G_ROWS // GCH

    @functools.partial(
        pl.kernel,
        out_type=jax.ShapeDtypeStruct((M_PAD, HID2), jnp.int32),
        mesh=plsc.VectorSubcoreMesh(**_SC_MESH),
        scratch_types=[
            pltpu.VMEM((NCH, GCH), jnp.int32),
            pltpu.VMEM((16,), jnp.int32),
            pltpu.VMEM((2, GCH, HID2), jnp.int32),
            pltpu.SemaphoreType.DMA,
            pltpu.SemaphoreType.DMA,
            pltpu.SemaphoreType.DMA,
            pltpu.SemaphoreType.DMA,
        ],
    )
    def k(x_hbm, tor_hbm, rm_hbm, xs_hbm, idx_v, rm_v, rows_v, g0, g1, w0, w1):
        wid = _sc_wid()
        gsem = (g0, g1)
        wsem = (w0, w1)
        pltpu.sync_copy(tor_hbm.at[pl.ds(wid * NCH, NCH)], idx_v)
        pltpu.sync_copy(rm_hbm, rm_v)
        rtot = rm_v[pl.ds(0, 16)][0]

        def pred(c):
            return wid * G_ROWS + c * GCH < rtot

        @pl.when(pred(0))
        def _():
            pltpu.async_copy(x_hbm.at[idx_v.at[0]], rows_v.at[0], gsem[0])

        for c in range(NCH):
            b = c % 2
            nb = (c + 1) % 2
            if c + 1 < NCH:
                @pl.when(pred(c + 1))
                def _(c=c, b=b, nb=nb):
                    if c >= 1:
                        pltpu.make_async_copy(rows_v.at[nb],
                                              xs_hbm.at[pl.ds(0, GCH)],
                                              wsem[nb]).wait()
                    pltpu.async_copy(x_hbm.at[idx_v.at[c + 1]],
                                     rows_v.at[nb], gsem[nb])

            @pl.when(pred(c))
            def _(c=c, b=b):
                pltpu.make_async_copy(x_hbm.at[idx_v.at[c]], rows_v.at[b],
                                      gsem[b]).wait()
                pltpu.async_copy(
                    rows_v.at[b],
                    xs_hbm.at[pl.ds(wid * G_ROWS + c * GCH, GCH)],
                    wsem[b])

        for b in range(2):
            last = NCH - 2 + b
            @pl.when(pred(last))
            def _(b=b):
                pltpu.make_async_copy(rows_v.at[b],
                                      xs_hbm.at[pl.ds(0, GCH)],
                                      wsem[b]).wait()

    return k(xb32, tor2, rmeta)


C_TOK = 4
C_PER_W = S // NW
N_CH = C_PER_W // C_TOK
_UNROLL = 4


def _combine(x, ys, p02, p12, rw0, rw1, zw):
    @functools.partial(
        pl.kernel,
        out_type=jax.ShapeDtypeStruct((S, HIDDEN), jnp.float32),
        mesh=plsc.VectorSubcoreMesh(**_SC_MESH),
        scratch_types=[
            pltpu.VMEM((N_CH, C_TOK), jnp.int32),
            pltpu.VMEM((N_CH, C_TOK), jnp.int32),
            pltpu.VMEM((C_PER_W,), jnp.float32),
            pltpu.VMEM((C_PER_W,), jnp.float32),
            pltpu.VMEM((C_PER_W,), jnp.float32),
            pltpu.VMEM((2, C_TOK, HIDDEN), jnp.float32),
            pltpu.VMEM((2, C_TOK, HIDDEN), jnp.float32),
            pltpu.VMEM((2, C_TOK, HIDDEN), jnp.float32),
            pltpu.VMEM((2, C_TOK, HIDDEN), jnp.float32),
            pltpu.SemaphoreType.DMA,
            pltpu.SemaphoreType.DMA,
            pltpu.SemaphoreType.DMA,
            pltpu.SemaphoreType.DMA,
        ],
    )
    def k(x_hbm, ys_hbm, p0_hbm, p1_hbm, rw0_hbm, rw1_hbm, zw_hbm, out_hbm,
          p0_v, p1_v, rw0_v, rw1_v, zw_v, x_b, y0_b, y1_b, o_b,
          gs0, gs1, ws0, ws1):
        gsem = (gs0, gs1)
        wsem = (ws0, ws1)
        wid = _sc_wid()
        base = wid * C_PER_W
        pltpu.sync_copy(p0_hbm.at[pl.ds(wid * N_CH, N_CH)], p0_v)
        pltpu.sync_copy(p1_hbm.at[pl.ds(wid * N_CH, N_CH)], p1_v)
        pltpu.sync_copy(rw0_hbm.at[pl.ds(base, C_PER_W)], rw0_v)
        pltpu.sync_copy(rw1_hbm.at[pl.ds(base, C_PER_W)], rw1_v)
        pltpu.sync_copy(zw_hbm.at[pl.ds(base, C_PER_W)], zw_v)

        def fire(c):
            b = c % 2
            pltpu.async_copy(ys_hbm.at[p0_v.at[c]], y0_b.at[b], gsem[b])
            pltpu.async_copy(ys_hbm.at[p1_v.at[c]], y1_b.at[b], gsem[b])
            pltpu.async_copy(x_hbm.at[pl.ds(base + c * C_TOK, C_TOK)],
                             x_b.at[b], gsem[b])

        def drain_in(c):
            b = c % 2
            pltpu.make_async_copy(ys_hbm.at[p0_v.at[c]], y0_b.at[b],
                                  gsem[b]).wait()
            pltpu.make_async_copy(ys_hbm.at[p1_v.at[c]], y1_b.at[b],
                                  gsem[b]).wait()
            pltpu.make_async_copy(x_hbm.at[pl.ds(base, C_TOK)], x_b.at[b],
                                  gsem[b]).wait()

        fire(0)
        for c in range(N_CH):
            b = c % 2
            if c >= 2:
                pltpu.make_async_copy(o_b.at[b], out_hbm.at[pl.ds(0, C_TOK)],
                                      wsem[b]).wait()
            if c + 1 < N_CH:
                fire(c + 1)
            drain_in(c)
            for j in range(C_TOK):
                q = c * C_TOK + j
                qs, ql = (q // 16) * 16, q % 16
                w0 = rw0_v[pl.ds(qs, 16)][ql]
                w1 = rw1_v[pl.ds(qs, 16)][ql]
                wz = zw_v[pl.ds(qs, 16)][ql]

                def lane(kk, carry, b=b, j=j, w0=w0, w1=w1, wz=wz):
                    for u in range(_UNROLL):
                        sl = pl.ds(kk * (16 * _UNROLL) + u * 16, 16)
                        o_b[b, j, sl] = (wz * x_b[b, j, sl]
                                         + w0 * y0_b[b, j, sl]
                                         + w1 * y1_b[b, j, sl])
                    return carry

                lax.fori_loop(0, HIDDEN // (16 * _UNROLL), lane, 0)
            pltpu.async_copy(o_b.at[b],
                             out_hbm.at[pl.ds(base + c * C_TOK, C_TOK)],
                             wsem[b])
        pltpu.make_async_copy(o_b.at[0], out_hbm.at[pl.ds(0, C_TOK)],
                              wsem[0]).wait()
        pltpu.make_async_copy(o_b.at[1], out_hbm.at[pl.ds(0, C_TOK)],
                              wsem[1]).wait()

    return k(x, ys, p02, p12, rw0, rw1, zw)


def _dispatch(e0, e1):
    keys = jnp.concatenate([e0, e1])
    order = jnp.argsort(keys)
    ks = keys[order]
    first = jnp.searchsorted(ks, ks, side="left")
    rank_sorted = jnp.arange(2 * S, dtype=jnp.int32) - first.astype(jnp.int32)
    rank = jnp.zeros(2 * S, jnp.int32).at[order].set(rank_sorted)
    counts = jnp.bincount(keys, length=N_ROUTED + 1).astype(jnp.int32)
    rc = ((counts[:N_ROUTED] + TM - 1) // TM) * TM
    offsets = jnp.concatenate([jnp.zeros(1, jnp.int32), jnp.cumsum(rc)])
    valid = keys < N_ROUTED
    pos = jnp.where(valid, offsets[jnp.minimum(keys, N_ROUTED - 1)] + rank, 0)
    token_of_row = jnp.zeros(M_PAD, jnp.int32).at[
        jnp.where(valid, pos, M_PAD - 1)].set(
        jnp.where(valid, jnp.arange(2 * S, dtype=jnp.int32) % S, 0))
    tile_start = offsets // TM
    t_iota = jnp.arange(N_TILES)
    te = jnp.sum(tile_start[None, 1:N_ROUTED] <= t_iota[:, None], axis=1)
    te = jnp.where(t_iota < tile_start[N_ROUTED], te, -1).astype(jnp.int32)
    rtot = jnp.broadcast_to(offsets[N_ROUTED], (16,)).astype(jnp.int32)
    return pos, token_of_row, te, rtot


def kernel(hidden_states, router_weight, e_score_correction_bias, w_gate, w_up, w_down):
    x = hidden_states.reshape(-1, HIDDEN).astype(jnp.float32)
    rwt_pad = jnp.zeros((HIDDEN, EPAD), jnp.float32).at[:, :N_EXP].set(router_weight.T)
    bias_pad = jnp.full((8, EPAD), -1e30, jnp.float32).at[:, :N_EXP].set(
        e_score_correction_bias[None, :])
    r, xb = _router(x, rwt_pad, bias_pad)
    e0 = r[:, 0].astype(jnp.int32)
    e1 = r[:, 1].astype(jnp.int32)
    rw0 = r[:, 2]
    rw1 = r[:, 3]
    zw = r[:, 4]

    pos, token_of_row, te, rtot = _dispatch(e0, e1)
    xs32 = _gather_rows(xb, token_of_row.reshape(M_PAD // GCH, GCH), rtot)
    ys = _gmm(te, xs32, w_gate, w_up, w_down)
    p02 = pos[:S].reshape(S // C_TOK, C_TOK)
    p12 = pos[S:].reshape(S // C_TOK, C_TOK)
    out = _combine(x, ys, p02, p12, rw0, rw1, zw)
    return out.reshape(B, S, HIDDEN)


# speedup vs baseline: 1.5120x; 1.0781x over previous
import functools

import jax
import jax.numpy as jnp
from jax import lax
from jax.experimental import pallas as pl
from jax.experimental.pallas import tpu as pltpu
from jax.experimental.pallas import tpu_sc as plsc

B = 1
S = 2048
HIDDEN = 2048
FFN = 512
N_ROUTED = 64
N_EXP = 80
EPAD = 128
TOPK = 2
SCALE = 1.0
EPS = 1e-20

HID2 = HIDDEN // 2
TM = 64
M_PAD = 8192
N_TILES = M_PAD // TM
S_TILE = 256


def _pack2(x):
    c = x.shape[1] // 2
    lo = jax.lax.bitcast_convert_type(x[:, :c].astype(jnp.bfloat16), jnp.uint16)
    hi = jax.lax.bitcast_convert_type(x[:, c:].astype(jnp.bfloat16), jnp.uint16)
    w = lo.astype(jnp.uint32) | (hi.astype(jnp.uint32) << 16)
    return jax.lax.bitcast_convert_type(w, jnp.int32)


def _unpack2(w):
    u = jax.lax.bitcast_convert_type(w, jnp.uint32)
    lo = jax.lax.bitcast_convert_type((u & 0xFFFF).astype(jnp.uint16),
                                      jnp.bfloat16)
    hi = jax.lax.bitcast_convert_type((u >> 16).astype(jnp.uint16),
                                      jnp.bfloat16)
    return jnp.concatenate([lo, hi], axis=1)


def _router_body(x_ref, rwt_ref, bias_ref, out_ref, xb_ref):
    xb_ref[...] = _pack2(x_ref[...])
    logits = jax.lax.dot_general(
        x_ref[...], rwt_ref[...], (((1,), (0,)), ((), ())),
        preferred_element_type=jnp.float32)
    scores = jax.nn.sigmoid(logits)
    c = scores + bias_ref[0:1, :]
    iota = jax.lax.broadcasted_iota(jnp.int32, (S_TILE, EPAD), 1)
    m0 = jnp.max(c, axis=1, keepdims=True)
    a0 = jnp.min(jnp.where(c == m0, iota, EPAD), axis=1, keepdims=True)
    c1 = jnp.where(iota == a0, -1e30, c)
    m1 = jnp.max(c1, axis=1, keepdims=True)
    a1 = jnp.min(jnp.where(c1 == m1, iota, EPAD), axis=1, keepdims=True)
    s0 = jnp.sum(jnp.where(iota == a0, scores, 0.0), axis=1, keepdims=True)
    s1 = jnp.sum(jnp.where(iota == a1, scores, 0.0), axis=1, keepdims=True)
    den = s0 + s1 + EPS
    w0 = s0 / den * SCALE
    w1 = s1 / den * SCALE
    z0 = a0 >= N_ROUTED
    z1 = a1 >= N_ROUTED
    rw0 = jnp.where(z0, 0.0, w0)
    rw1 = jnp.where(z1, 0.0, w1)
    zw = jnp.where(z0, w0, 0.0) + jnp.where(z1, w1, 0.0)
    e0 = jnp.where(z0, N_ROUTED, a0).astype(jnp.float32)
    e1 = jnp.where(z1, N_ROUTED, a1).astype(jnp.float32)
    li = jax.lax.broadcasted_iota(jnp.int32, (S_TILE, EPAD), 1)
    out = jnp.where(li == 0, e0, 0.0)
    out = jnp.where(li == 1, e1, out)
    out = jnp.where(li == 2, rw0, out)
    out = jnp.where(li == 3, rw1, out)
    out = jnp.where(li == 4, zw, out)
    out_ref[...] = out


def _router(x, rwt_pad, bias_pad):
    return pl.pallas_call(
        _router_body,
        grid=(S // S_TILE,),
        in_specs=[
            pl.BlockSpec((S_TILE, HIDDEN), lambda i: (i, 0)),
            pl.BlockSpec((HIDDEN, EPAD), lambda i: (0, 0)),
            pl.BlockSpec((8, EPAD), lambda i: (0, 0)),
        ],
        out_specs=[pl.BlockSpec((S_TILE, EPAD), lambda i: (i, 0)),
                   pl.BlockSpec((S_TILE, HID2), lambda i: (i, 0))],
        out_shape=[jax.ShapeDtypeStruct((S, EPAD), jnp.float32),
                   jax.ShapeDtypeStruct((S, HID2), jnp.int32)],
    )(x, rwt_pad, bias_pad)


def _gmm_body(te_ref, xs_ref, wg_ref, wu_ref, wd_ref, ys_ref):
    t = pl.program_id(0)
    e = te_ref[t]

    @pl.when(e >= 0)
    def _():
        xt = _unpack2(xs_ref[...])
        wg = wg_ref[0].astype(jnp.bfloat16)
        wu = wu_ref[0].astype(jnp.bfloat16)
        wd = wd_ref[0].astype(jnp.bfloat16)
        g = jax.lax.dot_general(xt, wg, (((1,), (1,)), ((), ())),
                                preferred_element_type=jnp.float32)
        u = jax.lax.dot_general(xt, wu, (((1,), (1,)), ((), ())),
                                preferred_element_type=jnp.float32)
        h = (g * jax.nn.sigmoid(g) * u).astype(jnp.bfloat16)
        y = jax.lax.dot_general(h, wd, (((1,), (1,)), ((), ())),
                                preferred_element_type=jnp.float32)
        ys_ref[...] = y

    @pl.when(jnp.logical_and(e < 0, t == 0))
    def _():
        ys_ref[...] = jnp.zeros_like(ys_ref)


def _gmm(te, xs, w_gate, w_up, w_down):
    def clamp(e):
        return jnp.clip(e, 0, N_ROUTED - 1)

    grid_spec = pltpu.PrefetchScalarGridSpec(
        num_scalar_prefetch=1,
        grid=(N_TILES,),
        in_specs=[
            pl.BlockSpec((TM, HID2), lambda t, te: (t, 0)),
            pl.BlockSpec((1, FFN, HIDDEN), lambda t, te: (clamp(te[t]), 0, 0)),
            pl.BlockSpec((1, FFN, HIDDEN), lambda t, te: (clamp(te[t]), 0, 0)),
            pl.BlockSpec((1, HIDDEN, FFN), lambda t, te: (clamp(te[t]), 0, 0)),
        ],
        out_specs=pl.BlockSpec((TM, HIDDEN), lambda t, te: (t, 0)),
    )
    return pl.pallas_call(
        _gmm_body,
        grid_spec=grid_spec,
        out_shape=jax.ShapeDtypeStruct((M_PAD, HIDDEN), jnp.float32),
    )(te, xs, w_gate, w_up, w_down)


NW = 32
S_PER_W = 2 * S // NW
SCH = 32
SNCH = S_PER_W // SCH
_SC_MESH = dict(core_axis_name="c", subcore_axis_name="s")


def _sc_wid():
    return lax.axis_index("s") * 2 + lax.axis_index("c")


def _scatter_rows(xb32, pos_s2):
    @functools.partial(
        pl.kernel,
        out_type=jax.ShapeDtypeStruct((M_PAD, HID2), jnp.int32),
        mesh=plsc.VectorSubcoreMesh(**_SC_MESH),
        scratch_types=[
            pltpu.VMEM((SNCH, SCH), jnp.int32),
            pltpu.VMEM((2, SCH, HID2), jnp.int32),
            pltpu.SemaphoreType.DMA,
            pltpu.SemaphoreType.DMA,
            pltpu.SemaphoreType.DMA,
            pltpu.SemaphoreType.DMA,
        ],
    )
    def k(x_hbm, ps_hbm, xs_hbm, idx_v, rows_v, g0, g1, w0, w1):
        wid = _sc_wid()
        gsem = (g0, g1)
        wsem = (w0, w1)
        tokbase = (wid * S_PER_W) % S
        pltpu.sync_copy(ps_hbm.at[pl.ds(wid * SNCH, SNCH)], idx_v)
        pltpu.async_copy(x_hbm.at[pl.ds(tokbase, SCH)], rows_v.at[0], gsem[0])
        for c in range(SNCH):
            b = c % 2
            nb = (c + 1) % 2
            if c + 1 < SNCH:
                if c >= 1:
                    pltpu.make_async_copy(rows_v.at[nb],
                                          xs_hbm.at[pl.ds(0, SCH)],
                                          wsem[nb]).wait()
                pltpu.async_copy(x_hbm.at[pl.ds(tokbase + (c + 1) * SCH, SCH)],
                                 rows_v.at[nb], gsem[nb])
            pltpu.make_async_copy(x_hbm.at[pl.ds(tokbase, SCH)], rows_v.at[b],
                                  gsem[b]).wait()
            pltpu.async_copy(rows_v.at[b], xs_hbm.at[idx_v.at[c]], wsem[b])
        for b in range(2):
            last = SNCH - 2 + b
            if last >= 0:
                pltpu.make_async_copy(rows_v.at[b], xs_hbm.at[pl.ds(0, SCH)],
                                      wsem[b]).wait()

    return k(xb32, pos_s2)


C_TOK = 4
C_PER_W = S // NW
N_CH = C_PER_W // C_TOK
_UNROLL = 4


def _combine(x, ys, p02, p12, rw0, rw1, zw):
    @functools.partial(
        pl.kernel,
        out_type=jax.ShapeDtypeStruct((S, HIDDEN), jnp.float32),
        mesh=plsc.VectorSubcoreMesh(**_SC_MESH),
        scratch_types=[
            pltpu.VMEM((N_CH, C_TOK), jnp.int32),
            pltpu.VMEM((N_CH, C_TOK), jnp.int32),
            pltpu.VMEM((C_PER_W,), jnp.float32),
            pltpu.VMEM((C_PER_W,), jnp.float32),
            pltpu.VMEM((C_PER_W,), jnp.float32),
            pltpu.VMEM((2, C_TOK, HIDDEN), jnp.float32),
            pltpu.VMEM((2, C_TOK, HIDDEN), jnp.float32),
            pltpu.VMEM((2, C_TOK, HIDDEN), jnp.float32),
            pltpu.VMEM((2, C_TOK, HIDDEN), jnp.float32),
            pltpu.SemaphoreType.DMA,
            pltpu.SemaphoreType.DMA,
            pltpu.SemaphoreType.DMA,
            pltpu.SemaphoreType.DMA,
        ],
    )
    def k(x_hbm, ys_hbm, p0_hbm, p1_hbm, rw0_hbm, rw1_hbm, zw_hbm, out_hbm,
          p0_v, p1_v, rw0_v, rw1_v, zw_v, x_b, y0_b, y1_b, o_b,
          gs0, gs1, ws0, ws1):
        gsem = (gs0, gs1)
        wsem = (ws0, ws1)
        wid = _sc_wid()
        base = wid * C_PER_W
        pltpu.sync_copy(p0_hbm.at[pl.ds(wid * N_CH, N_CH)], p0_v)
        pltpu.sync_copy(p1_hbm.at[pl.ds(wid * N_CH, N_CH)], p1_v)
        pltpu.sync_copy(rw0_hbm.at[pl.ds(base, C_PER_W)], rw0_v)
        pltpu.sync_copy(rw1_hbm.at[pl.ds(base, C_PER_W)], rw1_v)
        pltpu.sync_copy(zw_hbm.at[pl.ds(base, C_PER_W)], zw_v)

        def fire(c):
            b = c % 2
            pltpu.async_copy(ys_hbm.at[p0_v.at[c]], y0_b.at[b], gsem[b])
            pltpu.async_copy(ys_hbm.at[p1_v.at[c]], y1_b.at[b], gsem[b])
            pltpu.async_copy(x_hbm.at[pl.ds(base + c * C_TOK, C_TOK)],
                             x_b.at[b], gsem[b])

        def drain_in(c):
            b = c % 2
            pltpu.make_async_copy(ys_hbm.at[p0_v.at[c]], y0_b.at[b],
                                  gsem[b]).wait()
            pltpu.make_async_copy(ys_hbm.at[p1_v.at[c]], y1_b.at[b],
                                  gsem[b]).wait()
            pltpu.make_async_copy(x_hbm.at[pl.ds(base, C_TOK)], x_b.at[b],
                                  gsem[b]).wait()

        fire(0)
        for c in range(N_CH):
            b = c % 2
            if c >= 2:
                pltpu.make_async_copy(o_b.at[b], out_hbm.at[pl.ds(0, C_TOK)],
                                      wsem[b]).wait()
            if c + 1 < N_CH:
                fire(c + 1)
            drain_in(c)
            for j in range(C_TOK):
                q = c * C_TOK + j
                qs, ql = (q // 16) * 16, q % 16
                w0 = rw0_v[pl.ds(qs, 16)][ql]
                w1 = rw1_v[pl.ds(qs, 16)][ql]
                wz = zw_v[pl.ds(qs, 16)][ql]

                def lane(kk, carry, b=b, j=j, w0=w0, w1=w1, wz=wz):
                    for u in range(_UNROLL):
                        sl = pl.ds(kk * (16 * _UNROLL) + u * 16, 16)
                        o_b[b, j, sl] = (wz * x_b[b, j, sl]
                                         + w0 * y0_b[b, j, sl]
                                         + w1 * y1_b[b, j, sl])
                    return carry

                lax.fori_loop(0, HIDDEN // (16 * _UNROLL), lane, 0)
            pltpu.async_copy(o_b.at[b],
                             out_hbm.at[pl.ds(base + c * C_TOK, C_TOK)],
                             wsem[b])
        pltpu.make_async_copy(o_b.at[0], out_hbm.at[pl.ds(0, C_TOK)],
                              wsem[0]).wait()
        pltpu.make_async_copy(o_b.at[1], out_hbm.at[pl.ds(0, C_TOK)],
                              wsem[1]).wait()

    return k(x, ys, p02, p12, rw0, rw1, zw)


def _dispatch(e0, e1):
    keys = jnp.concatenate([e0, e1])
    order = jnp.argsort(keys)
    ks = keys[order]
    first = jnp.searchsorted(ks, ks, side="left")
    rank_sorted = jnp.arange(2 * S, dtype=jnp.int32) - first.astype(jnp.int32)
    rank = jnp.zeros(2 * S, jnp.int32).at[order].set(rank_sorted)
    counts = jnp.bincount(keys, length=N_ROUTED + 1).astype(jnp.int32)
    rc = ((counts[:N_ROUTED] + TM - 1) // TM) * TM
    offsets = jnp.concatenate([jnp.zeros(1, jnp.int32), jnp.cumsum(rc)])
    valid = keys < N_ROUTED
    pos = offsets[jnp.minimum(keys, N_ROUTED - 1)] + rank
    pos_g = jnp.where(valid, pos, 0)
    pos_s = jnp.where(valid, pos, M_PAD - 1)
    tile_start = offsets // TM
    t_iota = jnp.arange(N_TILES)
    te = jnp.sum(tile_start[None, 1:N_ROUTED] <= t_iota[:, None], axis=1)
    te = jnp.where(t_iota < tile_start[N_ROUTED], te, -1).astype(jnp.int32)
    return pos_g, pos_s, te


def kernel(hidden_states, router_weight, e_score_correction_bias, w_gate, w_up, w_down):
    x = hidden_states.reshape(-1, HIDDEN).astype(jnp.float32)
    rwt_pad = jnp.zeros((HIDDEN, EPAD), jnp.float32).at[:, :N_EXP].set(router_weight.T)
    bias_pad = jnp.full((8, EPAD), -1e30, jnp.float32).at[:, :N_EXP].set(
        e_score_correction_bias[None, :])
    r, xb = _router(x, rwt_pad, bias_pad)
    e0 = r[:, 0].astype(jnp.int32)
    e1 = r[:, 1].astype(jnp.int32)
    rw0 = r[:, 2]
    rw1 = r[:, 3]
    zw = r[:, 4]

    pos_g, pos_s, te = _dispatch(e0, e1)
    xs32 = _scatter_rows(xb, pos_s.reshape(2 * S // SCH, SCH))
    ys = _gmm(te, xs32, w_gate, w_up, w_down)
    p02 = pos_g[:S].reshape(S // C_TOK, C_TOK)
    p12 = pos_g[S:].reshape(S // C_TOK, C_TOK)
    out = _combine(x, ys, p02, p12, rw0, rw1, zw)
    return out.reshape(B, S, HIDDEN)
